# baseline (speedup 1.0000x reference)
import jax
import jax.numpy as jnp
from jax import lax
from jax.experimental import pallas as pl
from jax.experimental.pallas import tpu as pltpu
from jax.experimental.pallas import tpu_sc as plsc

_NC = 2
_NS = 16
_NW = _NC * _NS
_CHUNK = 128
_NB = 4
_ROWB = 1000


def _make_sc_agg(n, feat, e, compute_deg):
    cpw = -(-e // (_NW * _CHUNK))
    cpw = ((cpw + _NB - 1) // _NB) * _NB
    npad = n + 8
    npad16 = ((npad + 15) // 16) * 16
    span = (n // _NS) & ~7
    tail = n - span * _NS
    mesh = plsc.VectorSubcoreMesh(core_axis_name="c", subcore_axis_name="s",
                                  num_cores=_NC, num_subcores=_NS)

    outs = [jax.ShapeDtypeStruct((_NC, n, feat), jnp.float32)]
    scratch = [
        pltpu.VMEM((cpw, _CHUNK), jnp.int32),
        pltpu.VMEM((cpw, _CHUNK), jnp.int32),
        pltpu.VMEM_SHARED((npad, feat), jnp.float32),
    ]
    if compute_deg:
        outs.append(jax.ShapeDtypeStruct((_NW, 1, npad16), jnp.float32))
        scratch.append(pltpu.VMEM((npad16,), jnp.float32))
    scratch += [pltpu.VMEM((_CHUNK, feat), jnp.float32)] * _NB
    scratch += [pltpu.SemaphoreType.DMA] * (2 * _NB)

    def body(p_hbm, src_hbm, dst_hbm, zeros_hbm, *rest):
        if compute_deg:
            out_hbm, deg_hbm, sidx_all, didx_all, acc_sh, ldeg = rest[:6]
            bufs = rest[6:]
        else:
            out_hbm, sidx_all, didx_all, acc_sh = rest[:4]
            bufs = rest[4:]
        rows = bufs[:_NB]
        gsem = bufs[_NB:2 * _NB]
        ssem = bufs[2 * _NB:3 * _NB]
        c = lax.axis_index("c")
        s = lax.axis_index("s")
        wid = s * _NC + c

        pltpu.sync_copy(src_hbm.at[pl.ds(wid * cpw, cpw)], sidx_all)
        pltpu.sync_copy(dst_hbm.at[pl.ds(wid * cpw, cpw)], didx_all)

        @pl.when(s == 0)
        def _():
            pltpu.sync_copy(zeros_hbm, acc_sh)

        if compute_deg:
            z16 = jnp.zeros((16,), jnp.float32)

            def zb(i, carry):
                ldeg[pl.ds(i * 16, 16)] = z16
                return carry

            lax.fori_loop(0, npad16 // 16, zb, 0)

        plsc.subcore_barrier()

        for b in range(_NB):
            pltpu.async_copy(p_hbm.at[sidx_all.at[b]], rows[b], gsem[b])

        if compute_deg:
            ones16 = jnp.ones((16,), jnp.float32)

            def db(q, carry):
                for r in range(_CHUNK // 16):
                    d = didx_all[q, pl.ds(r * 16, 16)]
                    plsc.addupdate_scatter(ldeg, [d], ones16)
                return carry

            lax.fori_loop(0, cpw, db, 0)

        def outer(k, carry):
            for b in range(_NB):
                i = k * _NB + b
                pltpu.make_async_copy(
                    p_hbm.at[sidx_all.at[i]], rows[b], gsem[b]).wait()
                pltpu.async_copy(
                    rows[b], acc_sh.at[didx_all.at[i]], ssem[b], add=True)
                pltpu.make_async_copy(
                    rows[b], acc_sh.at[didx_all.at[i]], ssem[b]).wait()

                @pl.when(i + _NB < cpw)
                def _():
                    pltpu.async_copy(
                        p_hbm.at[sidx_all.at[i + _NB]], rows[b], gsem[b])
            return carry

        lax.fori_loop(0, cpw // _NB, outer, 0)

        if compute_deg:
            pltpu.sync_copy(ldeg, deg_hbm.at[wid, 0])
        plsc.subcore_barrier()
        pltpu.sync_copy(
            acc_sh.at[pl.ds(s * span, span)],
            out_hbm.at[c, pl.ds(s * span, span)],
        )
        if tail:
            @pl.when(s == 0)
            def _():
                pltpu.sync_copy(
                    acc_sh.at[pl.ds(span * _NS, tail)],
                    out_hbm.at[c, pl.ds(span * _NS, tail)],
                )

    return pl.kernel(body, out_type=tuple(outs), mesh=mesh,
                     scratch_types=scratch,
                     compiler_params=pltpu.CompilerParams(
                         needs_layout_passes=False,
                         use_tc_tiling_on_sc=False))


def _tc_proj(x, wl, wr):
    n, d = x.shape
    h = wl.shape[0]
    dn = (((1,), (1,)), ((), ()))

    def body(x_ref, wl_ref, wr_ref, p_ref, r_ref):
        xv = x_ref[...]
        p_ref[...] = lax.dot_general(xv, wl_ref[...], dn,
                                     preferred_element_type=jnp.float32)
        r_ref[...] = lax.dot_general(xv, wr_ref[...], dn,
                                     preferred_element_type=jnp.float32)

    return pl.pallas_call(
        body,
        grid=(n // _ROWB,),
        in_specs=[
            pl.BlockSpec((_ROWB, d), lambda i: (i, 0)),
            pl.BlockSpec((h, d), lambda i: (0, 0)),
            pl.BlockSpec((h, d), lambda i: (0, 0)),
        ],
        out_specs=[pl.BlockSpec((_ROWB, h), lambda i: (i, 0))] * 2,
        out_shape=[jax.ShapeDtypeStruct((n, h), jnp.float32)] * 2,
    )(x, wl, wr)


def _tc_mid(a0, a1, degt, r1, b1, wl, wr):
    n, h = a0.shape
    nw = degt.shape[1]
    dn = (((1,), (1,)), ((), ()))

    def body(a0_ref, a1_ref, deg_ref, r_ref, b_ref, wl_ref, wr_ref,
             p_ref, rr_ref):
        deg = jnp.maximum(jnp.sum(deg_ref[...], axis=1, keepdims=True), 1.0)
        h1 = jnp.maximum(
            (a0_ref[...] + a1_ref[...]) / deg + b_ref[...] + r_ref[...], 0.0)
        p_ref[...] = lax.dot_general(h1, wl_ref[...], dn,
                                     preferred_element_type=jnp.float32)
        rr_ref[...] = lax.dot_general(h1, wr_ref[...], dn,
                                      preferred_element_type=jnp.float32)

    return pl.pallas_call(
        body,
        grid=(n // _ROWB,),
        in_specs=[
            pl.BlockSpec((_ROWB, h), lambda i: (i, 0)),
            pl.BlockSpec((_ROWB, h), lambda i: (i, 0)),
            pl.BlockSpec((_ROWB, nw), lambda i: (i, 0)),
            pl.BlockSpec((_ROWB, h), lambda i: (i, 0)),
            pl.BlockSpec((1, h), lambda i: (0, 0)),
            pl.BlockSpec((h, h), lambda i: (0, 0)),
            pl.BlockSpec((h, h), lambda i: (0, 0)),
        ],
        out_specs=[pl.BlockSpec((_ROWB, h), lambda i: (i, 0))] * 2,
        out_shape=[jax.ShapeDtypeStruct((n, h), jnp.float32)] * 2,
    )(a0, a1, degt, r1, b1, wl, wr)


def _tc_pool(a0, a1, degt, r2, b2, batch2, wo, bo, g):
    n, h = a0.shape
    nw = degt.shape[1]
    t = wo.shape[0]
    nblk = n // _ROWB
    dn1 = (((1,), (1,)), ((), ()))
    dn0 = (((0,), (0,)), ((), ()))

    def body(a0_ref, a1_ref, deg_ref, r_ref, b_ref, bat_ref, wo_ref, bo_ref,
             out_ref, accv, accc):
        i = pl.program_id(0)

        @pl.when(i == 0)
        def _():
            accv[...] = jnp.zeros_like(accv)
            accc[...] = jnp.zeros_like(accc)

        deg = jnp.maximum(jnp.sum(deg_ref[...], axis=1, keepdims=True), 1.0)
        h2 = jnp.maximum(
            (a0_ref[...] + a1_ref[...]) / deg + b_ref[...] + r_ref[...], 0.0)
        v = lax.dot_general(h2, wo_ref[...], dn1,
                            preferred_element_type=jnp.float32)
        gid = lax.broadcasted_iota(jnp.int32, (_ROWB, g), 1)
        mask = (bat_ref[...] == gid).astype(jnp.float32)
        accv[...] += lax.dot_general(mask, v, dn0,
                                     preferred_element_type=jnp.float32)
        accc[...] += lax.dot_general(mask, jnp.ones((_ROWB, 1), jnp.float32),
                                     dn0, preferred_element_type=jnp.float32)

        @pl.when(i == nblk - 1)
        def _():
            out_ref[...] = accv[...] / jnp.maximum(accc[...], 1.0) + bo_ref[...]

    return pl.pallas_call(
        body,
        grid=(nblk,),
        in_specs=[
            pl.BlockSpec((_ROWB, h), lambda i: (i, 0)),
            pl.BlockSpec((_ROWB, h), lambda i: (i, 0)),
            pl.BlockSpec((_ROWB, nw), lambda i: (i, 0)),
            pl.BlockSpec((_ROWB, h), lambda i: (i, 0)),
            pl.BlockSpec((1, h), lambda i: (0, 0)),
            pl.BlockSpec((_ROWB, 1), lambda i: (i, 0)),
            pl.BlockSpec((t, h), lambda i: (0, 0)),
            pl.BlockSpec((1, t), lambda i: (0, 0)),
        ],
        out_specs=pl.BlockSpec((g, t), lambda i: (0, 0)),
        out_shape=jax.ShapeDtypeStruct((g, t), jnp.float32),
        scratch_shapes=[
            pltpu.VMEM((g, t), jnp.float32),
            pltpu.VMEM((g, 1), jnp.float32),
        ],
    )(a0, a1, degt, r2, b2, batch2, wo, bo)


def kernel(x, edge_index, batch, W1_l, b1, W1_r, W2_l, b2, W2_r, Wo, bo):
    n, d = x.shape
    h = W1_l.shape[0]
    e = edge_index.shape[1]
    g = 256
    cpw = -(-e // (_NW * _CHUNK))
    cpw = ((cpw + _NB - 1) // _NB) * _NB
    pad = _NW * cpw * _CHUNK - e
    npad16 = ((n + 8 + 15) // 16) * 16
    src = jnp.concatenate([edge_index[0], jnp.zeros((pad,), jnp.int32)])
    src = src.reshape(_NW * cpw, _CHUNK)
    dst = jnp.concatenate([edge_index[1], jnp.full((pad,), n, jnp.int32)])
    dst = dst.reshape(_NW * cpw, _CHUNK)
    zeros_nf = jnp.zeros((n + 8, h), jnp.float32)

    p1, r1 = _tc_proj(x, W1_l, W1_r)

    agg1, deg_raw = _make_sc_agg(n, h, e, True)(p1, src, dst, zeros_nf)
    degt = deg_raw.reshape(_NW, npad16)[:, :n].T

    p2, r2 = _tc_mid(agg1[0], agg1[1], degt, r1, b1.reshape(1, h), W2_l, W2_r)

    (agg2,) = _make_sc_agg(n, h, e, False)(p2, src, dst, zeros_nf)

    return _tc_pool(agg2[0], agg2[1], degt, r2, b2.reshape(1, h),
                    batch.reshape(n, 1), Wo, bo.reshape(1, -1), g)


# speedup vs baseline: 2.2568x; 2.2568x over previous
import jax
import jax.numpy as jnp
from jax import lax
from jax.experimental import pallas as pl
from jax.experimental.pallas import tpu as pltpu
from jax.experimental.pallas import tpu_sc as plsc

_NC = 2
_NS = 16
_NW = _NC * _NS
_CHUNK = 128
_NB = 4
_ROWB = 1000


def _make_sc_agg(n, feat, e, compute_deg):
    hh = feat // 2
    cpw = -(-e // (_NS * _CHUNK))
    cpw = ((cpw + _NB - 1) // _NB) * _NB
    hcw = cpw // 2
    npad = n + 8
    npad16 = ((npad + 15) // 16) * 16
    span = (n // _NS) & ~7
    tail = n - span * _NS
    mesh = plsc.VectorSubcoreMesh(core_axis_name="c", subcore_axis_name="s",
                                  num_cores=_NC, num_subcores=_NS)

    outs = [jax.ShapeDtypeStruct((_NC, n, hh), jnp.float32)]
    scratch = [
        pltpu.VMEM((cpw, _CHUNK), jnp.int32),
        pltpu.VMEM((cpw, _CHUNK), jnp.int32),
        pltpu.VMEM_SHARED((npad, hh), jnp.float32),
        pltpu.VMEM_SHARED((n, hh), jnp.float32),
    ]
    if compute_deg:
        outs.append(jax.ShapeDtypeStruct((_NW, 1, npad16), jnp.float32))
        scratch.append(pltpu.VMEM((npad16,), jnp.float32))
    scratch += [pltpu.VMEM((_CHUNK, hh), jnp.float32)] * _NB
    scratch += [pltpu.SemaphoreType.DMA] * (2 * _NB)

    def body(p_hbm, src_hbm, dst_hbm, zeros_hbm, *rest):
        if compute_deg:
            out_hbm, deg_hbm, sidx_all, didx_all, acc_sh, tbl_sh, ldeg = rest[:7]
            bufs = rest[7:]
        else:
            out_hbm, sidx_all, didx_all, acc_sh, tbl_sh = rest[:5]
            bufs = rest[5:]
        rows = bufs[:_NB]
        gsem = bufs[_NB:2 * _NB]
        ssem = bufs[2 * _NB:3 * _NB]
        c = lax.axis_index("c")
        s = lax.axis_index("s")
        wid = s * _NC + c

        pltpu.sync_copy(src_hbm.at[pl.ds(s * cpw, cpw)], sidx_all)
        pltpu.sync_copy(dst_hbm.at[pl.ds(s * cpw, cpw)], didx_all)

        @pl.when(s == 0)
        def _():
            pltpu.sync_copy(zeros_hbm, acc_sh)

        @pl.when(s == 1)
        def _():
            pltpu.sync_copy(p_hbm.at[c], tbl_sh)

        if compute_deg:
            z16 = jnp.zeros((16,), jnp.float32)

            def zb(i, carry):
                ldeg[pl.ds(i * 16, 16)] = z16
                return carry

            lax.fori_loop(0, npad16 // 16, zb, 0)

        plsc.subcore_barrier()

        for b in range(_NB):
            pltpu.async_copy(tbl_sh.at[sidx_all.at[b]], rows[b], gsem[b])

        if compute_deg:
            ones16 = jnp.ones((16,), jnp.float32)

            def db(q, carry):
                for r in range(_CHUNK // 16):
                    d = didx_all[c * hcw + q, pl.ds(r * 16, 16)]
                    plsc.addupdate_scatter(ldeg, [d], ones16)
                return carry

            lax.fori_loop(0, hcw, db, 0)

        def outer(k, carry):
            for b in range(_NB):
                i = k * _NB + b
                pltpu.make_async_copy(
                    tbl_sh.at[sidx_all.at[i]], rows[b], gsem[b]).wait()
                pltpu.async_copy(
                    rows[b], acc_sh.at[didx_all.at[i]], ssem[b], add=True)
                pltpu.make_async_copy(
                    rows[b], acc_sh.at[didx_all.at[i]], ssem[b]).wait()

                @pl.when(i + _NB < cpw)
                def _():
                    pltpu.async_copy(
                        tbl_sh.at[sidx_all.at[i + _NB]], rows[b], gsem[b])
            return carry

        lax.fori_loop(0, cpw // _NB, outer, 0)

        if compute_deg:
            pltpu.sync_copy(ldeg, deg_hbm.at[wid, 0])
        plsc.subcore_barrier()
        pltpu.sync_copy(
            acc_sh.at[pl.ds(s * span, span)],
            out_hbm.at[c, pl.ds(s * span, span)],
        )
        if tail:
            @pl.when(s == 0)
            def _():
                pltpu.sync_copy(
                    acc_sh.at[pl.ds(span * _NS, tail)],
                    out_hbm.at[c, pl.ds(span * _NS, tail)],
                )

    return pl.kernel(body, out_type=tuple(outs), mesh=mesh,
                     scratch_types=scratch,
                     compiler_params=pltpu.CompilerParams(
                         needs_layout_passes=False,
                         use_tc_tiling_on_sc=False))


def _tc_proj(x, wl, wr):
    n, d = x.shape
    h = wl.shape[0]
    hh = h // 2
    dn = (((1,), (1,)), ((), ()))

    def body(x_ref, wl_ref, wr_ref, p_ref, r_ref):
        xv = x_ref[...]
        pv = lax.dot_general(xv, wl_ref[...], dn,
                             preferred_element_type=jnp.float32)
        p_ref[0, :, :] = pv[:, :hh]
        p_ref[1, :, :] = pv[:, hh:]
        r_ref[...] = lax.dot_general(xv, wr_ref[...], dn,
                                     preferred_element_type=jnp.float32)

    return pl.pallas_call(
        body,
        grid=(n // _ROWB,),
        in_specs=[
            pl.BlockSpec((_ROWB, d), lambda i: (i, 0)),
            pl.BlockSpec((h, d), lambda i: (0, 0)),
            pl.BlockSpec((h, d), lambda i: (0, 0)),
        ],
        out_specs=[pl.BlockSpec((2, _ROWB, hh), lambda i: (0, i, 0)),
                   pl.BlockSpec((_ROWB, h), lambda i: (i, 0))],
        out_shape=[jax.ShapeDtypeStruct((2, n, hh), jnp.float32),
                   jax.ShapeDtypeStruct((n, h), jnp.float32)],
    )(x, wl, wr)


def _tc_mid(a3, degt, r1, b1, wl, wr):
    n, h = r1.shape
    hh = h // 2
    nw = degt.shape[1]
    dn = (((1,), (1,)), ((), ()))

    def body(a_ref, deg_ref, r_ref, b_ref, wl_ref, wr_ref, p_ref, rr_ref):
        agg = jnp.concatenate([a_ref[0], a_ref[1]], axis=1)
        deg = jnp.maximum(jnp.sum(deg_ref[...], axis=1, keepdims=True), 1.0)
        h1 = jnp.maximum(agg / deg + b_ref[...] + r_ref[...], 0.0)
        pv = lax.dot_general(h1, wl_ref[...], dn,
                             preferred_element_type=jnp.float32)
        p_ref[0, :, :] = pv[:, :hh]
        p_ref[1, :, :] = pv[:, hh:]
        rr_ref[...] = lax.dot_general(h1, wr_ref[...], dn,
                                      preferred_element_type=jnp.float32)

    return pl.pallas_call(
        body,
        grid=(n // _ROWB,),
        in_specs=[
            pl.BlockSpec((2, _ROWB, hh), lambda i: (0, i, 0)),
            pl.BlockSpec((_ROWB, nw), lambda i: (i, 0)),
            pl.BlockSpec((_ROWB, h), lambda i: (i, 0)),
            pl.BlockSpec((1, h), lambda i: (0, 0)),
            pl.BlockSpec((h, h), lambda i: (0, 0)),
            pl.BlockSpec((h, h), lambda i: (0, 0)),
        ],
        out_specs=[pl.BlockSpec((2, _ROWB, hh), lambda i: (0, i, 0)),
                   pl.BlockSpec((_ROWB, h), lambda i: (i, 0))],
        out_shape=[jax.ShapeDtypeStruct((2, n, hh), jnp.float32),
                   jax.ShapeDtypeStruct((n, h), jnp.float32)],
    )(a3, degt, r1, b1, wl, wr)


def _tc_pool(a3, degt, r2, b2, batch2, wo, bo, g):
    n, h = r2.shape
    hh = h // 2
    nw = degt.shape[1]
    t = wo.shape[0]
    nblk = n // _ROWB
    dn1 = (((1,), (1,)), ((), ()))
    dn0 = (((0,), (0,)), ((), ()))

    def body(a_ref, deg_ref, r_ref, b_ref, bat_ref, wo_ref, bo_ref,
             out_ref, accv, accc):
        i = pl.program_id(0)

        @pl.when(i == 0)
        def _():
            accv[...] = jnp.zeros_like(accv)
            accc[...] = jnp.zeros_like(accc)

        agg = jnp.concatenate([a_ref[0], a_ref[1]], axis=1)
        deg = jnp.maximum(jnp.sum(deg_ref[...], axis=1, keepdims=True), 1.0)
        h2 = jnp.maximum(agg / deg + b_ref[...] + r_ref[...], 0.0)
        v = lax.dot_general(h2, wo_ref[...], dn1,
                            preferred_element_type=jnp.float32)
        gid = lax.broadcasted_iota(jnp.int32, (_ROWB, g), 1)
        mask = (bat_ref[...] == gid).astype(jnp.float32)
        accv[...] += lax.dot_general(mask, v, dn0,
                                     preferred_element_type=jnp.float32)
        accc[...] += lax.dot_general(mask, jnp.ones((_ROWB, 1), jnp.float32),
                                     dn0, preferred_element_type=jnp.float32)

        @pl.when(i == nblk - 1)
        def _():
            out_ref[...] = accv[...] / jnp.maximum(accc[...], 1.0) + bo_ref[...]

    return pl.pallas_call(
        body,
        grid=(nblk,),
        in_specs=[
            pl.BlockSpec((2, _ROWB, hh), lambda i: (0, i, 0)),
            pl.BlockSpec((_ROWB, nw), lambda i: (i, 0)),
            pl.BlockSpec((_ROWB, h), lambda i: (i, 0)),
            pl.BlockSpec((1, h), lambda i: (0, 0)),
            pl.BlockSpec((_ROWB, 1), lambda i: (i, 0)),
            pl.BlockSpec((t, h), lambda i: (0, 0)),
            pl.BlockSpec((1, t), lambda i: (0, 0)),
        ],
        out_specs=pl.BlockSpec((g, t), lambda i: (0, 0)),
        out_shape=jax.ShapeDtypeStruct((g, t), jnp.float32),
        scratch_shapes=[
            pltpu.VMEM((g, t), jnp.float32),
            pltpu.VMEM((g, 1), jnp.float32),
        ],
    )(a3, degt, r2, b2, batch2, wo, bo)


def kernel(x, edge_index, batch, W1_l, b1, W1_r, W2_l, b2, W2_r, Wo, bo):
    n, d = x.shape
    h = W1_l.shape[0]
    e = edge_index.shape[1]
    g = 256
    cpw = -(-e // (_NS * _CHUNK))
    cpw = ((cpw + _NB - 1) // _NB) * _NB
    pad = _NS * cpw * _CHUNK - e
    npad16 = ((n + 8 + 15) // 16) * 16
    src = jnp.concatenate([edge_index[0], jnp.zeros((pad,), jnp.int32)])
    src = src.reshape(_NS * cpw, _CHUNK)
    dst = jnp.concatenate([edge_index[1], jnp.full((pad,), n, jnp.int32)])
    dst = dst.reshape(_NS * cpw, _CHUNK)
    zeros_nf = jnp.zeros((n + 8, h // 2), jnp.float32)

    p1, r1 = _tc_proj(x, W1_l, W1_r)

    agg1, deg_raw = _make_sc_agg(n, h, e, True)(p1, src, dst, zeros_nf)
    degt = deg_raw.reshape(_NW, npad16)[:, :n].T

    p2, r2 = _tc_mid(agg1, degt, r1, b1.reshape(1, h), W2_l, W2_r)

    (agg2,) = _make_sc_agg(n, h, e, False)(p2, src, dst, zeros_nf)

    return _tc_pool(agg2, degt, r2, b2.reshape(1, h),
                    batch.reshape(n, 1), Wo, bo.reshape(1, -1), g)
